# baseline (speedup 1.0000x reference)
import dataclasses
import functools

import jax
import jax.numpy as jnp
from jax import lax
from jax.experimental import pallas as pl
from jax.experimental.pallas import tpu as pltpu
from jax.experimental.pallas import tpu_sc as plsc

_D = 128
_LANES = 16


def _expw_body(w_ref, o_ref):
    w = w_ref[...]
    o_ref[...] = jnp.exp(w - jnp.max(w))


def _sc_pool(q1, table, expw, B, L):
    NW = 32
    RPW = B // NW
    C = 40
    NCH = L // C
    NV = _D // _LANES
    mesh = plsc.VectorSubcoreMesh(core_axis_name="c", subcore_axis_name="s")
    cp = pltpu.CompilerParams()
    if "needs_layout_passes" in pltpu.CompilerParams.__dataclass_fields__:
        cp = dataclasses.replace(cp, needs_layout_passes=False)

    @functools.partial(
        pl.kernel,
        out_type=jax.ShapeDtypeStruct((B, _D), jnp.float32),
        mesh=mesh,
        compiler_params=cp,
        scratch_types=[
            pltpu.VMEM((RPW * L,), jnp.int32),
            pltpu.VMEM((L, _D), jnp.float32),
            pltpu.VMEM((L, _D), jnp.float32),
            pltpu.VMEM((L,), jnp.float32),
            pltpu.VMEM((L,), jnp.float32),
            pltpu.VMEM((RPW, _D), jnp.float32),
            pltpu.SemaphoreType.DMA,
            pltpu.SemaphoreType.DMA,
            pltpu.SemaphoreType.DMA,
            pltpu.SemaphoreType.DMA,
        ],
    )
    def run(q_hbm, t_hbm, ew_hbm, o_hbm, idx_v, rows_a, rows_b, w_a, w_b,
            out_v, sem_ea, sem_eb, sem_wa, sem_wb):
        wid = lax.axis_index("s") * 2 + lax.axis_index("c")
        base = wid * RPW
        pltpu.sync_copy(q_hbm.at[pl.ds(base * L, RPW * L)], idx_v)

        zero = jnp.zeros((_LANES,), jnp.float32)
        lane_iota = lax.iota(jnp.int32, _LANES)

        def issue(r, rows_buf, w_buf, sem_e, sem_w):
            for h in range(NCH):
                idx_h = idx_v.at[pl.ds(r * L + h * C, C)]
                pltpu.async_copy(t_hbm.at[idx_h], rows_buf.at[pl.ds(h * C, C)],
                                 sem_e)
                pltpu.async_copy(ew_hbm.at[idx_h], w_buf.at[pl.ds(h * C, C)],
                                 sem_w)

        def wait(rows_buf, w_buf, sem_e, sem_w):
            for h in range(NCH):
                idx_h = idx_v.at[pl.ds(h * C, C)]
                pltpu.make_async_copy(t_hbm.at[idx_h],
                                      rows_buf.at[pl.ds(h * C, C)],
                                      sem_e).wait()
                pltpu.make_async_copy(ew_hbm.at[idx_h],
                                      w_buf.at[pl.ds(h * C, C)],
                                      sem_w).wait()

        def compute(r, rows_buf, w_buf):
            def body(l, accs):
                sidx = jnp.full((_LANES,), l, dtype=jnp.int32)
                s = plsc.load_gather(w_buf, [sidx])
                return tuple(
                    accs[d] + s * rows_buf[l, pl.ds(d * _LANES, _LANES)]
                    for d in range(NV))

            accs = lax.fori_loop(0, L, body, tuple([zero] * NV), unroll=8)

            dsum = zero
            for j in range(L // _LANES):
                dsum = dsum + w_buf[pl.ds(j * _LANES, _LANES)]
            dsum = dsum + jnp.where(
                lane_iota >= _LANES - (L % _LANES),
                w_buf[pl.ds(L - _LANES, _LANES)], 0.0)
            w_buf[pl.ds(0, _LANES)] = plsc.cumsum(dsum)
            den = plsc.load_gather(
                w_buf, [jnp.full((_LANES,), _LANES - 1, jnp.int32)])
            rcp = 1.0 / den
            for d in range(NV):
                out_v[r, pl.ds(d * _LANES, _LANES)] = accs[d] * rcp

        issue(0, rows_a, w_a, sem_ea, sem_wa)

        @pl.loop(0, RPW, step=2)
        def _row(r):
            issue(r + 1, rows_b, w_b, sem_eb, sem_wb)
            wait(rows_a, w_a, sem_ea, sem_wa)
            compute(r, rows_a, w_a)

            @pl.when(r < RPW - 2)
            def _():
                issue(r + 2, rows_a, w_a, sem_ea, sem_wa)

            wait(rows_b, w_b, sem_eb, sem_wb)
            compute(r + 1, rows_b, w_b)

        pltpu.sync_copy(out_v, o_hbm.at[pl.ds(base, RPW)])

    return run(q1, table, expw)


def kernel(query, query_token_embeds_weight, weights_weight):
    B, L = query.shape
    V = query_token_embeds_weight.shape[0]
    q1 = query.astype(jnp.int32).reshape(B * L)
    w2d = weights_weight.reshape(V // 125, 125)
    expw2d = pl.pallas_call(
        _expw_body,
        out_shape=jax.ShapeDtypeStruct(w2d.shape, jnp.float32),
    )(w2d)
    expw = expw2d.reshape(V)
    return _sc_pool(q1, query_token_embeds_weight, expw, B, L)


# speedup vs baseline: 1.1655x; 1.1655x over previous
import dataclasses
import functools

import jax
import jax.numpy as jnp
from jax import lax
from jax.experimental import pallas as pl
from jax.experimental.pallas import tpu as pltpu
from jax.experimental.pallas import tpu_sc as plsc

_D = 128
_LANES = 16


def _expw_body(w_ref, o_ref):
    w = w_ref[...]
    o_ref[...] = jnp.exp(w - jnp.max(w))


def _sc_pool(q1, table, expw, B, L):
    NW = 32
    RPW = B // NW
    C = 40
    NCH = L // C
    NV = _D // _LANES
    mesh = plsc.VectorSubcoreMesh(core_axis_name="c", subcore_axis_name="s")
    cp = pltpu.CompilerParams()
    if "needs_layout_passes" in pltpu.CompilerParams.__dataclass_fields__:
        cp = dataclasses.replace(cp, needs_layout_passes=False)

    @functools.partial(
        pl.kernel,
        out_type=jax.ShapeDtypeStruct((B, _D), jnp.float32),
        mesh=mesh,
        compiler_params=cp,
        scratch_types=[
            pltpu.VMEM((RPW * L,), jnp.int32),
            pltpu.VMEM((L, _D), jnp.float32),
            pltpu.VMEM((L, _D), jnp.float32),
            pltpu.VMEM((L, _D), jnp.float32),
            pltpu.VMEM((L,), jnp.float32),
            pltpu.VMEM((L,), jnp.float32),
            pltpu.VMEM((L,), jnp.float32),
            pltpu.VMEM((RPW, _D), jnp.float32),
            pltpu.SemaphoreType.DMA,
            pltpu.SemaphoreType.DMA,
            pltpu.SemaphoreType.DMA,
        ],
    )
    def run(q_hbm, t_hbm, ew_hbm, o_hbm, idx_v, rows_a, rows_b, rows_c,
            w_a, w_b, w_c, out_v, sem_a, sem_b, sem_c):
        wid = lax.axis_index("s") * 2 + lax.axis_index("c")
        base = wid * RPW
        pltpu.sync_copy(q_hbm.at[pl.ds(base * L, RPW * L)], idx_v)

        rows_bufs = [rows_a, rows_b, rows_c]
        w_bufs = [w_a, w_b, w_c]
        sems = [sem_a, sem_b, sem_c]

        zero = jnp.zeros((_LANES,), jnp.float32)
        lane_iota = lax.iota(jnp.int32, _LANES)

        def issue(r, j):
            rows_buf, w_buf, sem = rows_bufs[j], w_bufs[j], sems[j]
            for h in range(NCH):
                idx_h = idx_v.at[pl.ds(r * L + h * C, C)]
                pltpu.async_copy(t_hbm.at[idx_h], rows_buf.at[pl.ds(h * C, C)],
                                 sem)
                pltpu.async_copy(ew_hbm.at[idx_h], w_buf.at[pl.ds(h * C, C)],
                                 sem)

        def wait(j):
            rows_buf, w_buf, sem = rows_bufs[j], w_bufs[j], sems[j]
            for h in range(NCH):
                idx_h = idx_v.at[pl.ds(h * C, C)]
                pltpu.make_async_copy(t_hbm.at[idx_h],
                                      rows_buf.at[pl.ds(h * C, C)],
                                      sem).wait()
                pltpu.make_async_copy(ew_hbm.at[idx_h],
                                      w_buf.at[pl.ds(h * C, C)],
                                      sem).wait()

        def compute(r, j):
            rows_buf, w_buf = rows_bufs[j], w_bufs[j]

            def body(l, accs):
                sidx = jnp.full((_LANES,), l, dtype=jnp.int32)
                s = plsc.load_gather(w_buf, [sidx])
                return tuple(
                    accs[d] + s * rows_buf[l, pl.ds(d * _LANES, _LANES)]
                    for d in range(NV))

            accs = lax.fori_loop(0, L, body, tuple([zero] * NV), unroll=2)

            dsum = zero
            for j in range(L // _LANES):
                dsum = dsum + w_buf[pl.ds(j * _LANES, _LANES)]
            dsum = dsum + jnp.where(
                lane_iota >= _LANES - (L % _LANES),
                w_buf[pl.ds(L - _LANES, _LANES)], 0.0)
            w_buf[pl.ds(0, _LANES)] = plsc.cumsum(dsum)
            den = plsc.load_gather(
                w_buf, [jnp.full((_LANES,), _LANES - 1, jnp.int32)])
            rcp = 1.0 / den
            for d in range(NV):
                out_v[r, pl.ds(d * _LANES, _LANES)] = accs[d] * rcp

        for j in range(3):
            issue(j, j)

        @pl.loop(0, RPW - (RPW % 3), step=3)
        def _row(r):
            for j in range(3):
                wait(j)
                compute(r + j, j)

                @pl.when(r + j + 3 < RPW)
                def _():
                    issue(r + j + 3, j)

        for k in range(RPW % 3):
            wait(k)
            compute(RPW - (RPW % 3) + k, k)

        pltpu.sync_copy(out_v, o_hbm.at[pl.ds(base, RPW)])

    return run(q1, table, expw)


def kernel(query, query_token_embeds_weight, weights_weight):
    B, L = query.shape
    V = query_token_embeds_weight.shape[0]
    q1 = query.astype(jnp.int32).reshape(B * L)
    w2d = weights_weight.reshape(V // 125, 125)
    expw2d = pl.pallas_call(
        _expw_body,
        out_shape=jax.ShapeDtypeStruct(w2d.shape, jnp.float32),
    )(w2d)
    expw = expw2d.reshape(V)
    return _sc_pool(q1, query_token_embeds_weight, expw, B, L)
